# baseline (speedup 1.0000x reference)
import functools

import jax
import jax.numpy as jnp
from jax import lax
from jax.experimental import pallas as pl
from jax.experimental.pallas import tpu as pltpu
from jax.experimental.pallas import tpu_sc as plsc

L = 16
NC = 2
NS = 16
NW = NC * NS
K = 32
SL = 8
K2 = 32


def _permute(v, idx):
  return jnp.take_along_axis(v, idx, axis=0, mode="promise_in_bounds")


def _butterfly_levels(lane):
  levels = []
  w = L
  while w > 1:
    h = w // 2
    fold = lane ^ h
    src = ((lane // (2 * h)) * w) + (lane % h)
    pick = (1 - ((lane // h) % 2)).astype(jnp.float32)
    levels.append((fold, src, pick))
    w = h
  return levels


_POS = (0, 8, 4, 12, 2, 10, 6, 14, 1, 9, 5, 13, 3, 11, 7, 15)


@functools.lru_cache(maxsize=None)
def _build(B):
  bpw = B // NW
  chunks = bpw // L
  mesh = plsc.VectorSubcoreMesh(core_axis_name="c", subcore_axis_name="s")

  @functools.partial(
      pl.kernel,
      out_type=jax.ShapeDtypeStruct((NC * L,), jnp.float32),
      mesh=mesh,
      compiler_params=pltpu.CompilerParams(needs_layout_passes=False),
      scratch_types=[
          pltpu.VMEM((bpw,), jnp.int32),
          pltpu.VMEM((bpw,), jnp.int32),
          pltpu.VMEM((bpw,), jnp.float32),
          pltpu.VMEM((L,), jnp.float32),
          pltpu.VMEM((bpw,), jnp.float32),
          pltpu.VMEM((bpw,), jnp.float32),
          pltpu.VMEM((2, L, SL, 128), jnp.float32),
          pltpu.VMEM((2, L, SL, 128), jnp.float32),
          pltpu.VMEM((L,), jnp.float32),
          pltpu.VMEM((NS * L,), jnp.float32),
          pltpu.VMEM_SHARED((NS * L,), jnp.float32),
          pltpu.SemaphoreType.DMA,
          pltpu.SemaphoreType.DMA,
          pltpu.SemaphoreType.DMA,
      ],
  )
  def sc_loss(u_hbm, i_hbm, r_hbm, a_hbm, bU_hbm, bI_hbm, gU_hbm, gI_hbm,
              out_hbm, idxu_v, idxi_v, r_v, a_v, bu_v, bi_v, gu_b, gi_b,
              accv, red_v, shared, sem, sem_g0, sem_g1):
    cid = lax.axis_index("c")
    sid = lax.axis_index("s")
    base = (cid * NS + sid) * bpw

    cps = [
        pltpu.async_copy(u_hbm.at[pl.ds(base, bpw)], idxu_v, sem),
        pltpu.async_copy(i_hbm.at[pl.ds(base, bpw)], idxi_v, sem),
        pltpu.async_copy(r_hbm.at[pl.ds(base, bpw)], r_v, sem),
        pltpu.async_copy(a_hbm, a_v, sem),
    ]
    for cp in cps:
      cp.wait()

    beta_cps = [
        pltpu.async_copy(bU_hbm.at[idxu_v], bu_v, sem),
        pltpu.async_copy(bI_hbm.at[idxi_v], bi_v, sem),
    ]

    sems = (sem_g0, sem_g1)

    def gather(c, par):
      row0 = c * L
      uvec = idxu_v[pl.ds(row0, L)]
      ivec = idxi_v[pl.ds(row0, L)]
      sg = sems[par]
      pltpu.async_copy(gU_hbm.at[uvec // K2], gu_b.at[par], sg)
      pltpu.async_copy(gI_hbm.at[ivec // K2], gi_b.at[par], sg)

    def drain(par):
      sg = sems[par]
      pltpu.make_async_copy(
          gU_hbm.at[pl.ds(0, L)], gu_b.at[par], sg).wait()
      pltpu.make_async_copy(
          gI_hbm.at[pl.ds(0, L)], gi_b.at[par], sg).wait()

    gather(0, 0)
    for cp in beta_cps:
      cp.wait()

    alpha_vec = a_v[...]
    lane = lax.iota(jnp.int32, L)
    levels = _butterfly_levels(lane)

    def compute(c, par, acc):
      row0 = c * L
      uvec = idxu_v[pl.ds(row0, L)]
      ivec = idxi_v[pl.ds(row0, L)]
      ps = []
      for s in range(L):
        j = _POS[s]
        ru = uvec[j] % K2
        ri = ivec[j] % K2
        su, cu = ru // 4, (ru % 4) * K
        si, ci = ri // 4, (ri % 4) * K
        ps.append(
            gu_b[par, j, su, pl.ds(cu, L)] * gi_b[par, j, si, pl.ds(ci, L)]
            + gu_b[par, j, su, pl.ds(cu + L, L)]
            * gi_b[par, j, si, pl.ds(ci + L, L)])
      for fold, src, pick in levels:
        nxt = []
        for m in range(0, len(ps), 2):
          af = ps[m] + _permute(ps[m], fold)
          bf = ps[m + 1] + _permute(ps[m + 1], fold)
          pa = _permute(af, src)
          pb = _permute(bf, src)
          nxt.append(pb + (pa - pb) * pick)
        ps = nxt
      dots = ps[0]
      pred = alpha_vec + bu_v[pl.ds(row0, L)] + bi_v[pl.ds(row0, L)] + dots
      d = pred - r_v[pl.ds(row0, L)]
      return acc + d * d

    def pair(h, acc):
      c0 = 2 * h
      gather(c0 + 1, 1)
      drain(0)
      acc = compute(c0, 0, acc)

      @pl.when(c0 + 2 < chunks)
      def _():
        gather(c0 + 2, 0)

      drain(1)
      return compute(c0 + 1, 1, acc)

    acc = lax.fori_loop(0, chunks // 2, pair, alpha_vec * 0.0)

    accv[...] = acc
    pltpu.sync_copy(accv, shared.at[pl.ds(sid * L, L)])
    plsc.subcore_barrier()

    @pl.when(sid == 0)
    def _():
      pltpu.sync_copy(shared, red_v)
      tot = red_v[pl.ds(0, L)]
      for s in range(1, NS):
        tot = tot + red_v[pl.ds(s * L, L)]
      for fold, _, _ in levels:
        tot = tot + _permute(tot, fold)
      accv[...] = tot * (0.5 / B)
      pltpu.sync_copy(accv, out_hbm.at[pl.ds(cid * L, L)])

  return sc_loss


def kernel(sampleU, sampleI, sampleR, alpha, betaU, betaI, gammaU, gammaI):
  B = sampleU.shape[0]
  alpha_vec = jnp.full((L,), alpha, jnp.float32)
  gU8 = gammaU.reshape(gammaU.shape[0] // K2, SL, 128)
  gI8 = gammaI.reshape(gammaI.shape[0] // K2, SL, 128)
  out = _build(B)(sampleU, sampleI, sampleR, alpha_vec,
                  betaU, betaI, gU8, gI8)
  return out[0] + out[L]


# speedup vs baseline: 2.3939x; 2.3939x over previous
import functools

import jax
import jax.numpy as jnp
from jax import lax
from jax.experimental import pallas as pl
from jax.experimental.pallas import tpu as pltpu
from jax.experimental.pallas import tpu_sc as plsc

L = 16
NC = 2
NS = 16
NW = NC * NS
K = 32
SL = 8


def _permute(v, idx):
  return jnp.take_along_axis(v, idx, axis=0, mode="promise_in_bounds")


def _butterfly_levels(lane):
  levels = []
  w = L
  while w > 1:
    h = w // 2
    fold = lane ^ h
    src = ((lane // (2 * h)) * w) + (lane % h)
    pick = (1 - ((lane // h) % 2)).astype(jnp.float32)
    levels.append((fold, src, pick))
    w = h
  return levels


_POS = (0, 8, 4, 12, 2, 10, 6, 14, 1, 9, 5, 13, 3, 11, 7, 15)


@functools.lru_cache(maxsize=None)
def _build(B):
  bpw = B // NW
  chunks = bpw // L
  mesh = plsc.VectorSubcoreMesh(core_axis_name="c", subcore_axis_name="s")

  @functools.partial(
      pl.kernel,
      out_type=jax.ShapeDtypeStruct((NC * L,), jnp.float32),
      mesh=mesh,
      compiler_params=pltpu.CompilerParams(needs_layout_passes=False),
      scratch_types=[
          pltpu.VMEM((bpw,), jnp.int32),
          pltpu.VMEM((bpw,), jnp.int32),
          pltpu.VMEM((bpw,), jnp.float32),
          pltpu.VMEM((L,), jnp.float32),
          pltpu.VMEM((bpw,), jnp.float32),
          pltpu.VMEM((bpw,), jnp.float32),
          pltpu.VMEM((2, L, SL, K), jnp.float32),
          pltpu.VMEM((2, L, SL, K), jnp.float32),
          pltpu.VMEM((L,), jnp.float32),
          pltpu.VMEM((NS * L,), jnp.float32),
          pltpu.VMEM_SHARED((NS * L,), jnp.float32),
          pltpu.SemaphoreType.DMA,
          pltpu.SemaphoreType.DMA,
          pltpu.SemaphoreType.DMA,
      ],
  )
  def sc_loss(u_hbm, i_hbm, r_hbm, a_hbm, bU_hbm, bI_hbm, gU_hbm, gI_hbm,
              out_hbm, idxu_v, idxi_v, r_v, a_v, bu_v, bi_v, gu_b, gi_b,
              accv, red_v, shared, sem, sem_g0, sem_g1):
    cid = lax.axis_index("c")
    sid = lax.axis_index("s")
    base = (cid * NS + sid) * bpw

    cps = [
        pltpu.async_copy(u_hbm.at[pl.ds(base, bpw)], idxu_v, sem),
        pltpu.async_copy(i_hbm.at[pl.ds(base, bpw)], idxi_v, sem),
        pltpu.async_copy(r_hbm.at[pl.ds(base, bpw)], r_v, sem),
        pltpu.async_copy(a_hbm, a_v, sem),
    ]
    for cp in cps:
      cp.wait()

    beta_cps = [
        pltpu.async_copy(bU_hbm.at[idxu_v], bu_v, sem),
        pltpu.async_copy(bI_hbm.at[idxi_v], bi_v, sem),
    ]

    sems = (sem_g0, sem_g1)

    def gather(c, par):
      row0 = c * L
      uvec = idxu_v[pl.ds(row0, L)]
      ivec = idxi_v[pl.ds(row0, L)]
      sg = sems[par]
      for j in range(L):
        pltpu.async_copy(gU_hbm.at[uvec[j] // SL], gu_b.at[par, j], sg)
        pltpu.async_copy(gI_hbm.at[ivec[j] // SL], gi_b.at[par, j], sg)

    def drain(par):
      sg = sems[par]
      pltpu.make_async_copy(gU_hbm.at[pl.ds(0, L)], gu_b.at[par], sg).wait()
      pltpu.make_async_copy(gI_hbm.at[pl.ds(0, L)], gi_b.at[par], sg).wait()

    gather(0, 0)
    for cp in beta_cps:
      cp.wait()

    alpha_vec = a_v[...]
    lane = lax.iota(jnp.int32, L)
    levels = _butterfly_levels(lane)

    def compute(c, par, acc):
      row0 = c * L
      uvec = idxu_v[pl.ds(row0, L)]
      ivec = idxi_v[pl.ds(row0, L)]
      ps = []
      for s in range(L):
        j = _POS[s]
        ju = uvec[j] % SL
        ji = ivec[j] % SL
        ps.append(
            gu_b[par, j, ju, pl.ds(0, L)] * gi_b[par, j, ji, pl.ds(0, L)]
            + gu_b[par, j, ju, pl.ds(L, L)] * gi_b[par, j, ji, pl.ds(L, L)])
      for fold, src, pick in levels:
        nxt = []
        for m in range(0, len(ps), 2):
          af = ps[m] + _permute(ps[m], fold)
          bf = ps[m + 1] + _permute(ps[m + 1], fold)
          pa = _permute(af, src)
          pb = _permute(bf, src)
          nxt.append(pb + (pa - pb) * pick)
        ps = nxt
      dots = ps[0]
      pred = alpha_vec + bu_v[pl.ds(row0, L)] + bi_v[pl.ds(row0, L)] + dots
      d = pred - r_v[pl.ds(row0, L)]
      return acc + d * d

    def pair(h, acc):
      c0 = 2 * h
      gather(c0 + 1, 1)
      drain(0)
      acc = compute(c0, 0, acc)

      @pl.when(c0 + 2 < chunks)
      def _():
        gather(c0 + 2, 0)

      drain(1)
      return compute(c0 + 1, 1, acc)

    acc = lax.fori_loop(0, chunks // 2, pair, alpha_vec * 0.0)

    accv[...] = acc
    pltpu.sync_copy(accv, shared.at[pl.ds(sid * L, L)])
    plsc.subcore_barrier()

    @pl.when(sid == 0)
    def _():
      pltpu.sync_copy(shared, red_v)
      tot = red_v[pl.ds(0, L)]
      for s in range(1, NS):
        tot = tot + red_v[pl.ds(s * L, L)]
      for fold, _, _ in levels:
        tot = tot + _permute(tot, fold)
      accv[...] = tot * (0.5 / B)
      pltpu.sync_copy(accv, out_hbm.at[pl.ds(cid * L, L)])

  return sc_loss


def kernel(sampleU, sampleI, sampleR, alpha, betaU, betaI, gammaU, gammaI):
  B = sampleU.shape[0]
  alpha_vec = jnp.full((L,), alpha, jnp.float32)
  gU3 = gammaU.reshape(gammaU.shape[0] // SL, SL, K)
  gI3 = gammaI.reshape(gammaI.shape[0] // SL, SL, K)
  out = _build(B)(sampleU, sampleI, sampleR, alpha_vec,
                  betaU, betaI, gU3, gI3)
  return out[0] + out[L]
